# baseline (speedup 1.0000x reference)
import functools

import jax
import jax.numpy as jnp
from jax import lax
from jax.experimental import pallas as pl
from jax.experimental.pallas import tpu as pltpu
from jax.experimental.pallas import tpu_sc as plsc

N = 10000
E = 320000
D = 128
NPAD = 10240
NW = 32
EPW = E // NW
CH = 128
NCHUNK = E // CH
P2 = 1024
TRASH = P2
REP = 4
A2BLK = P2 + CH
A2ROWS = REP * A2BLK
CCAP = 10240

_mesh = plsc.VectorSubcoreMesh(core_axis_name="c", subcore_axis_name="s")


@functools.partial(
    pl.kernel,
    out_type=(
        jax.ShapeDtypeStruct((NW, NPAD), jnp.float32),
        jax.ShapeDtypeStruct((NW, NPAD), jnp.float32),
        jax.ShapeDtypeStruct((NW, CCAP), jnp.int32),
        jax.ShapeDtypeStruct((NW, CCAP), jnp.int32),
        jax.ShapeDtypeStruct((NW, 16), jnp.int32),
    ),
    mesh=_mesh,
    compiler_params=pltpu.CompilerParams(needs_layout_passes=False),
    scratch_types=[
        pltpu.VMEM((EPW,), jnp.int32),
        pltpu.VMEM((EPW,), jnp.int32),
        pltpu.VMEM((NPAD,), jnp.float32),
        pltpu.VMEM((NPAD,), jnp.float32),
        pltpu.VMEM((CCAP,), jnp.int32),
        pltpu.VMEM((CCAP,), jnp.int32),
        pltpu.VMEM((16,), jnp.int32),
    ],
)
def _stage_a(src_hbm, dst_hbm, dsrc_out, ddst_out, csrc_out, cdst_out,
             cnt_out, src_v, dst_v, ds_v, dd_v, cs_v, cd_v, cnt_v):
    wid = lax.axis_index("s") * 2 + lax.axis_index("c")
    rep_off = lax.rem(wid, REP) * A2BLK
    e0 = wid * EPW
    pltpu.sync_copy(src_hbm.at[pl.ds(e0, EPW)], src_v)
    pltpu.sync_copy(dst_hbm.at[pl.ds(e0, EPW)], dst_v)

    zf = jnp.zeros((16,), jnp.float32)

    def zbody(i, carry):
        ds_v[pl.ds(i * 16, 16)] = zf
        dd_v[pl.ds(i * 16, 16)] = zf
        return carry

    lax.fori_loop(0, NPAD // 16, zbody, 0)

    ones = jnp.ones((16,), jnp.float32)

    def ebody(i, base):
        s = src_v[pl.ds(i * 16, 16)]
        t = dst_v[pl.ds(i * 16, 16)]
        plsc.addupdate_scatter(ds_v, [s], ones)
        plsc.addupdate_scatter(dd_v, [t], ones)
        m = t < P2
        inc = plsc.cumsum(m.astype(jnp.int32))
        pos = base + inc - 1
        plsc.store_scatter(cs_v, [pos], s, mask=m)
        plsc.store_scatter(cd_v, [pos], t + rep_off, mask=m)
        return base + plsc.all_reduce_population_count(m)

    cntv = lax.fori_loop(0, EPW // 16, ebody, jnp.zeros((16,), jnp.int32))

    iota = lax.iota(jnp.int32, 16)
    for j in range(CH // 16):
        pos = cntv + iota + 16 * j
        plsc.store_scatter(cs_v, [pos], jnp.zeros((16,), jnp.int32))
        plsc.store_scatter(cd_v, [pos], TRASH + rep_off + iota + 16 * j)

    cnt_v[...] = cntv
    pltpu.sync_copy(cnt_v, cnt_out.at[wid])
    pltpu.sync_copy(ds_v, dsrc_out.at[wid])
    pltpu.sync_copy(dd_v, ddst_out.at[wid])
    pltpu.sync_copy(cs_v, csrc_out.at[wid])
    pltpu.sync_copy(cd_v, cdst_out.at[wid])


NB = 2
NBE = 4
NCHT = CCAP // CH


def _zero_zbuf(zbuf):
    zf = jnp.zeros((16,), jnp.float32)

    def zb(i, carry):
        zbuf[i // 8, pl.ds((i % 8) * 16, 16)] = zf
        return carry

    lax.fori_loop(0, 32 * 8, zb, 0)


@functools.partial(
    pl.kernel,
    out_type=jax.ShapeDtypeStruct((2, NPAD, D), jnp.float32),
    mesh=_mesh,
    compiler_params=pltpu.CompilerParams(needs_layout_passes=False),
    scratch_types=[
        pltpu.VMEM((CH,), jnp.int32),
        pltpu.VMEM((CH,), jnp.int32),
        pltpu.VMEM((CH,), jnp.int32),
        pltpu.VMEM((CH,), jnp.int32),
        pltpu.VMEM((CH, D), jnp.float32),
        pltpu.VMEM((CH, D), jnp.float32),
        pltpu.VMEM((16, D), jnp.float32),
        pltpu.SemaphoreType.DMA,
        pltpu.SemaphoreType.DMA,
        pltpu.SemaphoreType.DMA,
        pltpu.SemaphoreType.DMA,
        pltpu.SemaphoreType.DMA,
        pltpu.VMEM_SHARED((NPAD, D), jnp.float32),
    ],
)
def _scat1(h_hbm, src_hbm, dst_hbm, out_hbm, si0, si1, di0, di1, r0b, r1b,
           zbuf, g0, g1, s0, s1, zsem, agg_sh):
    c = lax.axis_index("c")
    s = lax.axis_index("s")
    wid = s * 2 + c
    sidx = [si0, si1]
    didx = [di0, di1]
    rows = [r0b, r1b]
    gsem = [g0, g1]
    ssem = [s0, s1]

    zf = jnp.zeros((16,), jnp.float32)

    def zb(i, carry):
        zbuf[i // 8, pl.ds((i % 8) * 16, 16)] = zf
        return carry

    lax.fori_loop(0, 16 * 8, zb, 0)
    r0 = s * (NPAD // 16)
    zds = [pltpu.async_copy(zbuf, agg_sh.at[pl.ds(r0 + k * 16, 16)], zsem)
           for k in range((NPAD // 16) // 16)]
    for d in zds:
        d.wait()
    plsc.subcore_barrier()

    nloc = jnp.where(wid < NCHUNK - (NCHUNK // NW) * NW,
                     NCHUNK // NW + 1, NCHUNK // NW)

    def _half(i, par):
        npar = 1 - par

        @pl.when(i < nloc)
        def _issue(par=par):
            off = (wid + i * NW) * CH
            pltpu.sync_copy(src_hbm.at[pl.ds(off, CH)], sidx[par])

            @pl.when(i >= 2)
            def _drain_s():
                pltpu.make_async_copy(
                    h_hbm.at[pl.ds(0, CH)], rows[par], ssem[par]).wait()

            pltpu.async_copy(h_hbm.at[sidx[par]], rows[par], gsem[par])
            pltpu.sync_copy(dst_hbm.at[pl.ds(off, CH)], didx[par])

        @pl.when((i >= 1) & (i - 1 < nloc))
        def _finish(npar=npar):
            pltpu.make_async_copy(
                h_hbm.at[pl.ds(0, CH)], rows[npar], gsem[npar]).wait()
            pltpu.async_copy(rows[npar], agg_sh.at[didx[npar]],
                             ssem[npar], add=True)

    def pair(j, carry):
        _half(2 * j, 0)
        _half(2 * j + 1, 1)
        return carry

    lax.fori_loop(0, NCHT // 2, pair, 0)
    pltpu.make_async_copy(h_hbm.at[pl.ds(0, CH)], rows[0], ssem[0]).wait()
    pltpu.make_async_copy(h_hbm.at[pl.ds(0, CH)], rows[1], ssem[1]).wait()
    plsc.subcore_barrier()

    wds = [pltpu.async_copy(agg_sh.at[pl.ds(r0 + k * 64, 64)],
                            out_hbm.at[c, pl.ds(r0 + k * 64, 64)], zsem)
           for k in range((NPAD // 16) // 64)]
    for d in wds:
        d.wait()


@functools.partial(
    pl.kernel,
    out_type=jax.ShapeDtypeStruct((2, A2ROWS, D), jnp.float32),
    mesh=_mesh,
    compiler_params=pltpu.CompilerParams(needs_layout_passes=False),
    scratch_types=[
        pltpu.VMEM((CH,), jnp.int32),
        pltpu.VMEM((CH,), jnp.int32),
        pltpu.VMEM((CH,), jnp.int32),
        pltpu.VMEM((CH,), jnp.int32),
        pltpu.VMEM((CH, D), jnp.float32),
        pltpu.VMEM((CH, D), jnp.float32),
        pltpu.VMEM((16, D), jnp.float32),
        pltpu.SemaphoreType.DMA,
        pltpu.SemaphoreType.DMA,
        pltpu.SemaphoreType.DMA,
        pltpu.SemaphoreType.DMA,
        pltpu.SemaphoreType.DMA,
        pltpu.VMEM_SHARED((A2ROWS, D), jnp.float32),
        pltpu.VMEM((16,), jnp.int32),
    ],
)
def _scat2(h_hbm, csrc_hbm, cdst_hbm, cnt_hbm, out_hbm, si0, si1, di0,
           di1, r0b, r1b, zbuf, g0, g1, s0, s1, zsem, agg_sh, cnt_v):
    c = lax.axis_index("c")
    s = lax.axis_index("s")
    wid = s * 2 + c
    sidx = [si0, si1]
    didx = [di0, di1]
    rows = [r0b, r1b]
    gsem = [g0, g1]
    ssem = [s0, s1]

    pltpu.sync_copy(cnt_hbm.at[wid], cnt_v)
    _zero_zbuf(zbuf)
    r0 = s * (A2ROWS // 16)
    zds = [pltpu.async_copy(zbuf, agg_sh.at[pl.ds(r0 + k * 16, 16)], zsem)
           for k in range((A2ROWS // 16) // 16)]
    cnt = cnt_v[...][0]
    n = (cnt + CH - 1) // CH
    for d in zds:
        d.wait()
    plsc.subcore_barrier()

    e0 = wid * CCAP

    def _half(i, par):
        npar = 1 - par

        @pl.when(i < n)
        def _issue(par=par):
            off = e0 + i * CH
            pltpu.sync_copy(csrc_hbm.at[pl.ds(off, CH)], sidx[par])

            @pl.when(i >= 2)
            def _drain_s():
                pltpu.make_async_copy(
                    h_hbm.at[pl.ds(0, CH)], rows[par], ssem[par]).wait()

            pltpu.async_copy(h_hbm.at[sidx[par]], rows[par], gsem[par])
            pltpu.sync_copy(cdst_hbm.at[pl.ds(off, CH)], didx[par])

        @pl.when((i >= 1) & (i - 1 < n))
        def _finish(npar=npar):
            pltpu.make_async_copy(
                h_hbm.at[pl.ds(0, CH)], rows[npar], gsem[npar]).wait()
            pltpu.async_copy(rows[npar], agg_sh.at[didx[npar]],
                             ssem[npar], add=True)

    def pair(j, carry):
        _half(2 * j, 0)
        _half(2 * j + 1, 1)
        return carry

    lax.fori_loop(0, (n + 2) // 2, pair, 0)
    npar2 = lax.rem(n, 2)
    for b in range(2):
        @pl.when((n >= 2) & (npar2 == b))
        def _df1(b=b):
            pltpu.make_async_copy(h_hbm.at[pl.ds(0, CH)], rows[b],
                                  ssem[b]).wait()

        @pl.when((n >= 1) & (npar2 == 1 - b))
        def _df2(b=b):
            pltpu.make_async_copy(h_hbm.at[pl.ds(0, CH)], rows[b],
                                  ssem[b]).wait()
    plsc.subcore_barrier()

    wds = [pltpu.async_copy(agg_sh.at[pl.ds(r0 + k * 32, 32)],
                            out_hbm.at[c, pl.ds(r0 + k * 32, 32)], zsem)
           for k in range((A2ROWS // 16) // 32)]
    for d in wds:
        d.wait()


def _prep_body(feat_ref, dsp_ref, ddp_ref, h1s_ref, io_ref, ii_ref):
    dsrc = jnp.sum(dsp_ref[...], axis=0)
    ddst = jnp.sum(ddp_ref[...], axis=0)
    inv_out = lax.rsqrt(jnp.maximum(dsrc, 1.0))
    inv_in = lax.rsqrt(jnp.maximum(ddst, 1.0))
    io_ref[...] = inv_out[:, None]
    ii_ref[...] = inv_in[:, None]
    h1s_ref[...] = feat_ref[...] * inv_out[:, None]


_prep = pl.pallas_call(
    _prep_body,
    out_shape=(
        jax.ShapeDtypeStruct((NPAD, D), jnp.float32),
        jax.ShapeDtypeStruct((NPAD, 1), jnp.float32),
        jax.ShapeDtypeStruct((NPAD, 1), jnp.float32),
    ),
)


_RB = 1280


def _mm1_body(p_ref, ii_ref, io_ref, w_ref, b_ref, out_ref):
    agg = (p_ref[0] + p_ref[1]) * ii_ref[...]
    z = jnp.dot(agg, w_ref[...], preferred_element_type=jnp.float32)
    z = z + b_ref[...]
    h = jnp.where(z > 0, z, 0.01 * z)
    out_ref[...] = h * io_ref[...]


_mm1 = pl.pallas_call(
    _mm1_body,
    grid=(NPAD // _RB,),
    in_specs=[
        pl.BlockSpec((2, _RB, D), lambda i: (0, i, 0)),
        pl.BlockSpec((_RB, 1), lambda i: (i, 0)),
        pl.BlockSpec((_RB, 1), lambda i: (i, 0)),
        pl.BlockSpec((D, D), lambda i: (0, 0)),
        pl.BlockSpec((1, D), lambda i: (0, 0)),
    ],
    out_specs=pl.BlockSpec((_RB, D), lambda i: (i, 0)),
    out_shape=jax.ShapeDtypeStruct((NPAD, D), jnp.float32),
)


def _fin_body(p_ref, ii_ref, w2_ref, b2_ref, wl_ref, bl_ref, s_ref, out_ref):
    acc = p_ref[0, 0]
    for ci in range(2):
        for ri in range(REP):
            if (ci, ri) != (0, 0):
                acc = acc + p_ref[ci, ri]
    agg = acc * ii_ref[...]
    z = jnp.dot(agg, w2_ref[...], preferred_element_type=jnp.float32)
    z = z + b2_ref[...]
    emb = jnp.where(z > 0, z, 0.01 * z)
    pooled = jnp.sum(emb, axis=0, keepdims=True) * s_ref[...]
    out_ref[...] = (
        jnp.dot(pooled, wl_ref[...], preferred_element_type=jnp.float32)
        + bl_ref[...])


_fin = pl.pallas_call(
    _fin_body,
    out_shape=jax.ShapeDtypeStruct((1, D), jnp.float32),
)


def kernel(feat, edge_index, order, W1, b1, W2, b2, Wl, bl):
    src = edge_index[0]
    dst = edge_index[1]
    featp = jnp.pad(feat, ((0, NPAD - N), (0, 0)))

    dsp, ddp, csrc, cdst, cnts = _stage_a(src, dst)
    h1s, inv_out, inv_in = _prep(featp, dsp, ddp)
    p1 = _scat1(h1s, src, dst)
    h2s = _mm1(p1, inv_in, inv_out, W1, b1.reshape(1, D))
    p2 = _scat2(h2s, csrc.reshape(-1), cdst.reshape(-1), cnts)
    scale = jnp.ones((1, D), jnp.float32) / (
        jnp.asarray(order, jnp.float32) + 1.0)
    p2r = p2.reshape(2, REP, A2BLK, D)[:, :, :P2]
    out = _fin(p2r, inv_in[:P2], W2, b2.reshape(1, D), Wl,
               bl.reshape(1, D), scale)
    return out.reshape(D)


# speedup vs baseline: 1.0458x; 1.0458x over previous
import functools

import jax
import jax.numpy as jnp
from jax import lax
from jax.experimental import pallas as pl
from jax.experimental.pallas import tpu as pltpu
from jax.experimental.pallas import tpu_sc as plsc

N = 10000
E = 320000
D = 128
NPAD = 10240
NW = 32
EPW = E // NW
CH = 128
NCHUNK = E // CH
P2 = 1024
TRASH = P2
REP = 1
A2BLK = P2 + CH
A2ROWS = REP * A2BLK
CCAP = 10240

_mesh = plsc.VectorSubcoreMesh(core_axis_name="c", subcore_axis_name="s")


@functools.partial(
    pl.kernel,
    out_type=(
        jax.ShapeDtypeStruct((NW, NPAD), jnp.float32),
        jax.ShapeDtypeStruct((NW, NPAD), jnp.float32),
        jax.ShapeDtypeStruct((NW, CCAP), jnp.int32),
        jax.ShapeDtypeStruct((NW, CCAP), jnp.int32),
        jax.ShapeDtypeStruct((NW, 16), jnp.int32),
    ),
    mesh=_mesh,
    compiler_params=pltpu.CompilerParams(needs_layout_passes=False),
    scratch_types=[
        pltpu.VMEM((EPW,), jnp.int32),
        pltpu.VMEM((EPW,), jnp.int32),
        pltpu.VMEM((NPAD,), jnp.float32),
        pltpu.VMEM((NPAD,), jnp.float32),
        pltpu.VMEM((CCAP,), jnp.int32),
        pltpu.VMEM((CCAP,), jnp.int32),
        pltpu.VMEM((16,), jnp.int32),
    ],
)
def _stage_a(src_hbm, dst_hbm, dsrc_out, ddst_out, csrc_out, cdst_out,
             cnt_out, src_v, dst_v, ds_v, dd_v, cs_v, cd_v, cnt_v):
    wid = lax.axis_index("s") * 2 + lax.axis_index("c")
    rep_off = lax.rem(wid, REP) * A2BLK
    e0 = wid * EPW
    pltpu.sync_copy(src_hbm.at[pl.ds(e0, EPW)], src_v)
    pltpu.sync_copy(dst_hbm.at[pl.ds(e0, EPW)], dst_v)

    zf = jnp.zeros((16,), jnp.float32)

    def zbody(i, carry):
        ds_v[pl.ds(i * 16, 16)] = zf
        dd_v[pl.ds(i * 16, 16)] = zf
        return carry

    lax.fori_loop(0, NPAD // 16, zbody, 0)

    ones = jnp.ones((16,), jnp.float32)

    def ebody(i, base):
        s = src_v[pl.ds(i * 16, 16)]
        t = dst_v[pl.ds(i * 16, 16)]
        plsc.addupdate_scatter(ds_v, [s], ones)
        plsc.addupdate_scatter(dd_v, [t], ones)
        m = t < P2
        inc = plsc.cumsum(m.astype(jnp.int32))
        pos = base + inc - 1
        plsc.store_scatter(cs_v, [pos], s, mask=m)
        plsc.store_scatter(cd_v, [pos], t + rep_off, mask=m)
        return base + plsc.all_reduce_population_count(m)

    cntv = lax.fori_loop(0, EPW // 16, ebody, jnp.zeros((16,), jnp.int32))

    iota = lax.iota(jnp.int32, 16)
    for j in range(CH // 16):
        pos = cntv + iota + 16 * j
        plsc.store_scatter(cs_v, [pos], jnp.zeros((16,), jnp.int32))
        plsc.store_scatter(cd_v, [pos], TRASH + rep_off + iota + 16 * j)

    cnt_v[...] = cntv
    pltpu.sync_copy(cnt_v, cnt_out.at[wid])
    pltpu.sync_copy(ds_v, dsrc_out.at[wid])
    pltpu.sync_copy(dd_v, ddst_out.at[wid])
    pltpu.sync_copy(cs_v, csrc_out.at[wid])
    pltpu.sync_copy(cd_v, cdst_out.at[wid])


NB = 2
NBE = 4
NCHT = CCAP // CH


def _zero_zbuf(zbuf):
    zf = jnp.zeros((16,), jnp.float32)

    def zb(i, carry):
        zbuf[i // 8, pl.ds((i % 8) * 16, 16)] = zf
        return carry

    lax.fori_loop(0, 32 * 8, zb, 0)


@functools.partial(
    pl.kernel,
    out_type=jax.ShapeDtypeStruct((2, NPAD, D), jnp.float32),
    mesh=_mesh,
    compiler_params=pltpu.CompilerParams(needs_layout_passes=False),
    scratch_types=[
        pltpu.VMEM((CH,), jnp.int32),
        pltpu.VMEM((CH,), jnp.int32),
        pltpu.VMEM((CH,), jnp.int32),
        pltpu.VMEM((CH,), jnp.int32),
        pltpu.VMEM((CH,), jnp.int32),
        pltpu.VMEM((CH,), jnp.int32),
        pltpu.VMEM((CH, D), jnp.float32),
        pltpu.VMEM((CH, D), jnp.float32),
        pltpu.VMEM((16, D), jnp.float32),
        pltpu.SemaphoreType.DMA,
        pltpu.SemaphoreType.DMA,
        pltpu.SemaphoreType.DMA,
        pltpu.SemaphoreType.DMA,
        pltpu.SemaphoreType.DMA,
        pltpu.VMEM_SHARED((NPAD, D), jnp.float32),
    ],
)
def _scat1(h_hbm, pk_hbm, out_hbm, pk0, pk1, si0, si1, di0, di1, r0b, r1b,
           zbuf, g0, g1, s0, s1, zsem, agg_sh):
    c = lax.axis_index("c")
    s = lax.axis_index("s")
    wid = s * 2 + c
    pkb = [pk0, pk1]
    sidx = [si0, si1]
    didx = [di0, di1]
    rows = [r0b, r1b]
    gsem = [g0, g1]
    ssem = [s0, s1]

    zf = jnp.zeros((16,), jnp.float32)

    def zb(i, carry):
        zbuf[i // 8, pl.ds((i % 8) * 16, 16)] = zf
        return carry

    lax.fori_loop(0, 16 * 8, zb, 0)
    r0 = s * (NPAD // 16)
    zds = [pltpu.async_copy(zbuf, agg_sh.at[pl.ds(r0 + k * 16, 16)], zsem)
           for k in range((NPAD // 16) // 16)]
    for d in zds:
        d.wait()
    plsc.subcore_barrier()

    nloc = jnp.where(wid < NCHUNK - (NCHUNK // NW) * NW,
                     NCHUNK // NW + 1, NCHUNK // NW)

    def _half(i, par):
        npar = 1 - par

        @pl.when(i < nloc)
        def _issue(par=par):
            off = (wid + i * NW) * CH
            pltpu.sync_copy(pk_hbm.at[pl.ds(off, CH)], pkb[par])

            @pl.when(i >= 2)
            def _drain_s():
                pltpu.make_async_copy(
                    h_hbm.at[pl.ds(0, CH)], rows[par], ssem[par]).wait()

            for k in range(CH // 16):
                p = pkb[par][pl.ds(k * 16, 16)]
                sidx[par][pl.ds(k * 16, 16)] = p >> 14
                didx[par][pl.ds(k * 16, 16)] = p & 16383
            pltpu.async_copy(h_hbm.at[sidx[par]], rows[par], gsem[par])

        @pl.when((i >= 1) & (i - 1 < nloc))
        def _finish(npar=npar):
            pltpu.make_async_copy(
                h_hbm.at[pl.ds(0, CH)], rows[npar], gsem[npar]).wait()
            pltpu.async_copy(rows[npar], agg_sh.at[didx[npar]],
                             ssem[npar], add=True)

    def pair(j, carry):
        _half(2 * j, 0)
        _half(2 * j + 1, 1)
        return carry

    lax.fori_loop(0, NCHT // 2, pair, 0)
    pltpu.make_async_copy(h_hbm.at[pl.ds(0, CH)], rows[0], ssem[0]).wait()
    pltpu.make_async_copy(h_hbm.at[pl.ds(0, CH)], rows[1], ssem[1]).wait()
    plsc.subcore_barrier()

    wds = [pltpu.async_copy(agg_sh.at[pl.ds(r0 + k * 64, 64)],
                            out_hbm.at[c, pl.ds(r0 + k * 64, 64)], zsem)
           for k in range((NPAD // 16) // 64)]
    for d in wds:
        d.wait()


@functools.partial(
    pl.kernel,
    out_type=jax.ShapeDtypeStruct((2, A2ROWS, D), jnp.float32),
    mesh=_mesh,
    compiler_params=pltpu.CompilerParams(needs_layout_passes=False),
    scratch_types=[
        pltpu.VMEM((CH,), jnp.int32),
        pltpu.VMEM((CH,), jnp.int32),
        pltpu.VMEM((CH,), jnp.int32),
        pltpu.VMEM((CH,), jnp.int32),
        pltpu.VMEM((CH, D), jnp.float32),
        pltpu.VMEM((CH, D), jnp.float32),
        pltpu.VMEM((16, D), jnp.float32),
        pltpu.SemaphoreType.DMA,
        pltpu.SemaphoreType.DMA,
        pltpu.SemaphoreType.DMA,
        pltpu.SemaphoreType.DMA,
        pltpu.SemaphoreType.DMA,
        pltpu.VMEM_SHARED((A2ROWS, D), jnp.float32),
        pltpu.VMEM((16,), jnp.int32),
    ],
)
def _scat2(h_hbm, csrc_hbm, cdst_hbm, cnt_hbm, out_hbm, si0, si1, di0,
           di1, r0b, r1b, zbuf, g0, g1, s0, s1, zsem, agg_sh, cnt_v):
    c = lax.axis_index("c")
    s = lax.axis_index("s")
    wid = s * 2 + c
    sidx = [si0, si1]
    didx = [di0, di1]
    rows = [r0b, r1b]
    gsem = [g0, g1]
    ssem = [s0, s1]

    pltpu.sync_copy(cnt_hbm.at[wid], cnt_v)
    _zero_zbuf(zbuf)
    r0 = s * (A2ROWS // 16)
    zds = [pltpu.async_copy(zbuf, agg_sh.at[pl.ds(r0 + k * 16, 16)], zsem)
           for k in range((A2ROWS // 16) // 16)]
    zds.append(pltpu.async_copy(zbuf.at[pl.ds(0, 8)],
                                agg_sh.at[pl.ds(r0 + 64, 8)], zsem))
    cnt = cnt_v[...][0]
    n = (cnt + CH - 1) // CH
    for d in zds:
        d.wait()
    plsc.subcore_barrier()

    e0 = wid * CCAP

    def _half(i, par):
        npar = 1 - par

        @pl.when(i < n)
        def _issue(par=par):
            off = e0 + i * CH
            pltpu.sync_copy(csrc_hbm.at[pl.ds(off, CH)], sidx[par])

            @pl.when(i >= 2)
            def _drain_s():
                pltpu.make_async_copy(
                    h_hbm.at[pl.ds(0, CH)], rows[par], ssem[par]).wait()

            pltpu.async_copy(h_hbm.at[sidx[par]], rows[par], gsem[par])
            pltpu.sync_copy(cdst_hbm.at[pl.ds(off, CH)], didx[par])

        @pl.when((i >= 1) & (i - 1 < n))
        def _finish(npar=npar):
            pltpu.make_async_copy(
                h_hbm.at[pl.ds(0, CH)], rows[npar], gsem[npar]).wait()
            pltpu.async_copy(rows[npar], agg_sh.at[didx[npar]],
                             ssem[npar], add=True)

    def pair(j, carry):
        _half(2 * j, 0)
        _half(2 * j + 1, 1)
        return carry

    lax.fori_loop(0, (n + 2) // 2, pair, 0)
    npar2 = lax.rem(n, 2)
    for b in range(2):
        @pl.when((n >= 2) & (npar2 == b))
        def _df1(b=b):
            pltpu.make_async_copy(h_hbm.at[pl.ds(0, CH)], rows[b],
                                  ssem[b]).wait()

        @pl.when((n >= 1) & (npar2 == 1 - b))
        def _df2(b=b):
            pltpu.make_async_copy(h_hbm.at[pl.ds(0, CH)], rows[b],
                                  ssem[b]).wait()
    plsc.subcore_barrier()

    wds = [pltpu.async_copy(agg_sh.at[pl.ds(r0, 64)],
                            out_hbm.at[c, pl.ds(r0, 64)], zsem),
           pltpu.async_copy(agg_sh.at[pl.ds(r0 + 64, 8)],
                            out_hbm.at[c, pl.ds(r0 + 64, 8)], zsem)]
    for d in wds:
        d.wait()


def _prep_body(feat_ref, dsp_ref, ddp_ref, h1s_ref, io_ref, ii_ref):
    dsrc = jnp.sum(dsp_ref[...], axis=0)
    ddst = jnp.sum(ddp_ref[...], axis=0)
    inv_out = lax.rsqrt(jnp.maximum(dsrc, 1.0))
    inv_in = lax.rsqrt(jnp.maximum(ddst, 1.0))
    io_ref[...] = inv_out[:, None]
    ii_ref[...] = inv_in[:, None]
    h1s_ref[...] = feat_ref[...] * inv_out[:, None]


_prep = pl.pallas_call(
    _prep_body,
    out_shape=(
        jax.ShapeDtypeStruct((NPAD, D), jnp.float32),
        jax.ShapeDtypeStruct((NPAD, 1), jnp.float32),
        jax.ShapeDtypeStruct((NPAD, 1), jnp.float32),
    ),
)


_RB = 1280


def _mm1_body(p_ref, ii_ref, io_ref, w_ref, b_ref, out_ref):
    agg = (p_ref[0] + p_ref[1]) * ii_ref[...]
    z = jnp.dot(agg, w_ref[...], preferred_element_type=jnp.float32)
    z = z + b_ref[...]
    h = jnp.where(z > 0, z, 0.01 * z)
    out_ref[...] = h * io_ref[...]


_mm1 = pl.pallas_call(
    _mm1_body,
    grid=(NPAD // _RB,),
    in_specs=[
        pl.BlockSpec((2, _RB, D), lambda i: (0, i, 0)),
        pl.BlockSpec((_RB, 1), lambda i: (i, 0)),
        pl.BlockSpec((_RB, 1), lambda i: (i, 0)),
        pl.BlockSpec((D, D), lambda i: (0, 0)),
        pl.BlockSpec((1, D), lambda i: (0, 0)),
    ],
    out_specs=pl.BlockSpec((_RB, D), lambda i: (i, 0)),
    out_shape=jax.ShapeDtypeStruct((NPAD, D), jnp.float32),
)


def _fin_body(p_ref, ii_ref, w2_ref, b2_ref, wl_ref, bl_ref, s_ref, out_ref):
    acc = p_ref[0, 0]
    for ci in range(2):
        for ri in range(REP):
            if (ci, ri) != (0, 0):
                acc = acc + p_ref[ci, ri]
    agg = acc * ii_ref[...]
    z = jnp.dot(agg, w2_ref[...], preferred_element_type=jnp.float32)
    z = z + b2_ref[...]
    emb = jnp.where(z > 0, z, 0.01 * z)
    pooled = jnp.sum(emb, axis=0, keepdims=True) * s_ref[...]
    out_ref[...] = (
        jnp.dot(pooled, wl_ref[...], preferred_element_type=jnp.float32)
        + bl_ref[...])


_fin = pl.pallas_call(
    _fin_body,
    out_shape=jax.ShapeDtypeStruct((1, D), jnp.float32),
)


def kernel(feat, edge_index, order, W1, b1, W2, b2, Wl, bl):
    src = edge_index[0]
    dst = edge_index[1]
    featp = jnp.pad(feat, ((0, NPAD - N), (0, 0)))

    pk = (src << 14) | dst
    dsp, ddp, csrc, cdst, cnts = _stage_a(src, dst)
    h1s, inv_out, inv_in = _prep(featp, dsp, ddp)
    p1 = _scat1(h1s, pk)
    h2s = _mm1(p1, inv_in, inv_out, W1, b1.reshape(1, D))
    p2 = _scat2(h2s, csrc.reshape(-1), cdst.reshape(-1), cnts)
    scale = jnp.ones((1, D), jnp.float32) / (
        jnp.asarray(order, jnp.float32) + 1.0)
    p2r = p2.reshape(2, REP, A2BLK, D)[:, :, :P2]
    out = _fin(p2r, inv_in[:P2], W2, b2.reshape(1, D), Wl,
               bl.reshape(1, D), scale)
    return out.reshape(D)


# speedup vs baseline: 1.0474x; 1.0016x over previous
import functools

import jax
import jax.numpy as jnp
from jax import lax
from jax.experimental import pallas as pl
from jax.experimental.pallas import tpu as pltpu
from jax.experimental.pallas import tpu_sc as plsc

N = 10000
E = 320000
D = 128
NPAD = 10240
NW = 32
EPW = E // NW
CH = 128
NCHUNK = E // CH
P2 = 1024
TRASH = P2
REP = 1
A2BLK = P2 + CH
A2ROWS = REP * A2BLK
CCAP = 10240

_mesh = plsc.VectorSubcoreMesh(core_axis_name="c", subcore_axis_name="s")


@functools.partial(
    pl.kernel,
    out_type=(
        jax.ShapeDtypeStruct((NW, NPAD), jnp.float32),
        jax.ShapeDtypeStruct((NW, NPAD), jnp.float32),
        jax.ShapeDtypeStruct((NW, CCAP), jnp.int32),
        jax.ShapeDtypeStruct((NW, CCAP), jnp.int32),
        jax.ShapeDtypeStruct((NW, 16), jnp.int32),
    ),
    mesh=_mesh,
    compiler_params=pltpu.CompilerParams(needs_layout_passes=False),
    scratch_types=[
        pltpu.VMEM((EPW,), jnp.int32),
        pltpu.VMEM((EPW,), jnp.int32),
        pltpu.VMEM((NPAD,), jnp.float32),
        pltpu.VMEM((NPAD,), jnp.float32),
        pltpu.VMEM((CCAP,), jnp.int32),
        pltpu.VMEM((CCAP,), jnp.int32),
        pltpu.VMEM((16,), jnp.int32),
    ],
)
def _stage_a(src_hbm, dst_hbm, dsrc_out, ddst_out, csrc_out, cdst_out,
             cnt_out, src_v, dst_v, ds_v, dd_v, cs_v, cd_v, cnt_v):
    wid = lax.axis_index("s") * 2 + lax.axis_index("c")
    rep_off = lax.rem(wid, REP) * A2BLK
    e0 = wid * EPW
    pltpu.sync_copy(src_hbm.at[pl.ds(e0, EPW)], src_v)
    pltpu.sync_copy(dst_hbm.at[pl.ds(e0, EPW)], dst_v)

    zf = jnp.zeros((16,), jnp.float32)

    def zbody(i, carry):
        ds_v[pl.ds(i * 16, 16)] = zf
        dd_v[pl.ds(i * 16, 16)] = zf
        return carry

    lax.fori_loop(0, NPAD // 16, zbody, 0)

    ones = jnp.ones((16,), jnp.float32)

    def ebody(i, base):
        s = src_v[pl.ds(i * 16, 16)]
        t = dst_v[pl.ds(i * 16, 16)]
        plsc.addupdate_scatter(ds_v, [s], ones)
        plsc.addupdate_scatter(dd_v, [t], ones)
        m = t < P2
        inc = plsc.cumsum(m.astype(jnp.int32))
        pos = base + inc - 1
        plsc.store_scatter(cs_v, [pos], s, mask=m)
        plsc.store_scatter(cd_v, [pos], t + rep_off, mask=m)
        return base + plsc.all_reduce_population_count(m)

    cntv = lax.fori_loop(0, EPW // 16, ebody, jnp.zeros((16,), jnp.int32))

    iota = lax.iota(jnp.int32, 16)
    for j in range(CH // 16):
        pos = cntv + iota + 16 * j
        plsc.store_scatter(cs_v, [pos], jnp.zeros((16,), jnp.int32))
        plsc.store_scatter(cd_v, [pos], TRASH + rep_off + iota + 16 * j)

    cnt_v[...] = cntv
    pltpu.sync_copy(cnt_v, cnt_out.at[wid])
    pltpu.sync_copy(ds_v, dsrc_out.at[wid])
    pltpu.sync_copy(dd_v, ddst_out.at[wid])
    pltpu.sync_copy(cs_v, csrc_out.at[wid])
    pltpu.sync_copy(cd_v, cdst_out.at[wid])


NB = 2
NCHT = CCAP // CH


def _zero_zbuf(zbuf):
    zf = jnp.zeros((16,), jnp.float32)

    def zb(i, carry):
        zbuf[i // 8, pl.ds((i % 8) * 16, 16)] = zf
        return carry

    lax.fori_loop(0, 32 * 8, zb, 0)


@functools.partial(
    pl.kernel,
    out_type=jax.ShapeDtypeStruct((2, NPAD, D), jnp.float32),
    mesh=_mesh,
    compiler_params=pltpu.CompilerParams(needs_layout_passes=False),
    scratch_types=[
        pltpu.VMEM((CH,), jnp.int32),
        pltpu.VMEM((CH,), jnp.int32),
        pltpu.VMEM((CH,), jnp.int32),
        pltpu.VMEM((CH,), jnp.int32),
        pltpu.VMEM((CH,), jnp.int32),
        pltpu.VMEM((CH,), jnp.int32),
        pltpu.VMEM((CH, D), jnp.float32),
        pltpu.VMEM((CH, D), jnp.float32),
        pltpu.VMEM((16, D), jnp.float32),
        pltpu.SemaphoreType.DMA,
        pltpu.SemaphoreType.DMA,
        pltpu.SemaphoreType.DMA,
        pltpu.SemaphoreType.DMA,
        pltpu.SemaphoreType.DMA,
        pltpu.VMEM_SHARED((NPAD, D), jnp.float32),
    ],
)
def _scat1(h_hbm, pk_hbm, out_hbm, pk0, pk1, si0, si1, di0, di1, r0b, r1b,
           zbuf, g0, g1, s0, s1, zsem, agg_sh):
    c = lax.axis_index("c")
    s = lax.axis_index("s")
    wid = s * 2 + c
    pkb = [pk0, pk1]
    sidx = [si0, si1]
    didx = [di0, di1]
    rows = [r0b, r1b]
    gsem = [g0, g1]
    ssem = [s0, s1]

    zf = jnp.zeros((16,), jnp.float32)

    def zb(i, carry):
        zbuf[i // 8, pl.ds((i % 8) * 16, 16)] = zf
        return carry

    lax.fori_loop(0, 16 * 8, zb, 0)
    r0 = s * (NPAD // 16)
    zds = [pltpu.async_copy(zbuf, agg_sh.at[pl.ds(r0 + k * 16, 16)], zsem)
           for k in range((NPAD // 16) // 16)]
    for d in zds:
        d.wait()
    plsc.subcore_barrier()

    nloc = jnp.where(wid < NCHUNK - (NCHUNK // NW) * NW,
                     NCHUNK // NW + 1, NCHUNK // NW)

    def _half(i, par):
        npar = 1 - par

        @pl.when(i < nloc)
        def _issue(par=par):
            off = (wid + i * NW) * CH
            pltpu.sync_copy(pk_hbm.at[pl.ds(off, CH)], pkb[par])

            @pl.when(i >= 2)
            def _drain_s():
                pltpu.make_async_copy(
                    h_hbm.at[pl.ds(0, CH)], rows[par], ssem[par]).wait()

            for k in range(CH // 16):
                p = pkb[par][pl.ds(k * 16, 16)]
                sidx[par][pl.ds(k * 16, 16)] = p >> 14
                didx[par][pl.ds(k * 16, 16)] = p & 16383
            pltpu.async_copy(h_hbm.at[sidx[par]], rows[par], gsem[par])

        @pl.when((i >= 1) & (i - 1 < nloc))
        def _finish(npar=npar):
            pltpu.make_async_copy(
                h_hbm.at[pl.ds(0, CH)], rows[npar], gsem[npar]).wait()
            pltpu.async_copy(rows[npar], agg_sh.at[didx[npar]],
                             ssem[npar], add=True)

    def pair(j, carry):
        _half(2 * j, 0)
        _half(2 * j + 1, 1)
        return carry

    lax.fori_loop(0, NCHT // 2, pair, 0)
    pltpu.make_async_copy(h_hbm.at[pl.ds(0, CH)], rows[0], ssem[0]).wait()
    pltpu.make_async_copy(h_hbm.at[pl.ds(0, CH)], rows[1], ssem[1]).wait()
    plsc.subcore_barrier()

    wds = [pltpu.async_copy(agg_sh.at[pl.ds(r0 + k * 64, 64)],
                            out_hbm.at[c, pl.ds(r0 + k * 64, 64)], zsem)
           for k in range((NPAD // 16) // 64)]
    for d in wds:
        d.wait()


@functools.partial(
    pl.kernel,
    out_type=jax.ShapeDtypeStruct((2, A2ROWS, D), jnp.float32),
    mesh=_mesh,
    compiler_params=pltpu.CompilerParams(needs_layout_passes=False),
    scratch_types=[
        pltpu.VMEM((CH,), jnp.int32),
        pltpu.VMEM((CH,), jnp.int32),
        pltpu.VMEM((CH,), jnp.int32),
        pltpu.VMEM((CH,), jnp.int32),
        pltpu.VMEM((CH, D), jnp.float32),
        pltpu.VMEM((CH, D), jnp.float32),
        pltpu.VMEM((16, D), jnp.float32),
        pltpu.SemaphoreType.DMA,
        pltpu.SemaphoreType.DMA,
        pltpu.SemaphoreType.DMA,
        pltpu.SemaphoreType.DMA,
        pltpu.SemaphoreType.DMA,
        pltpu.VMEM_SHARED((A2ROWS, D), jnp.float32),
        pltpu.VMEM((16,), jnp.int32),
    ],
)
def _scat2(h_hbm, csrc_hbm, cdst_hbm, cnt_hbm, out_hbm, si0, si1, di0,
           di1, r0b, r1b, zbuf, g0, g1, s0, s1, zsem, agg_sh, cnt_v):
    c = lax.axis_index("c")
    s = lax.axis_index("s")
    wid = s * 2 + c
    sidx = [si0, si1]
    didx = [di0, di1]
    rows = [r0b, r1b]
    gsem = [g0, g1]
    ssem = [s0, s1]

    pltpu.sync_copy(cnt_hbm.at[wid], cnt_v)
    _zero_zbuf(zbuf)
    r0 = s * (A2ROWS // 16)
    zds = [pltpu.async_copy(zbuf, agg_sh.at[pl.ds(r0 + k * 16, 16)], zsem)
           for k in range((A2ROWS // 16) // 16)]
    zds.append(pltpu.async_copy(zbuf.at[pl.ds(0, 8)],
                                agg_sh.at[pl.ds(r0 + 64, 8)], zsem))
    cnt = cnt_v[...][0]
    n = (cnt + CH - 1) // CH
    for d in zds:
        d.wait()
    plsc.subcore_barrier()

    e0 = wid * CCAP

    def _half(i, par):
        npar = 1 - par

        @pl.when(i < n)
        def _issue(par=par):
            off = e0 + i * CH
            pltpu.sync_copy(csrc_hbm.at[pl.ds(off, CH)], sidx[par])

            @pl.when(i >= 2)
            def _drain_s():
                pltpu.make_async_copy(
                    h_hbm.at[pl.ds(0, CH)], rows[par], ssem[par]).wait()

            pltpu.async_copy(h_hbm.at[sidx[par]], rows[par], gsem[par])
            pltpu.sync_copy(cdst_hbm.at[pl.ds(off, CH)], didx[par])

        @pl.when((i >= 1) & (i - 1 < n))
        def _finish(npar=npar):
            pltpu.make_async_copy(
                h_hbm.at[pl.ds(0, CH)], rows[npar], gsem[npar]).wait()
            pltpu.async_copy(rows[npar], agg_sh.at[didx[npar]],
                             ssem[npar], add=True)

    def pair(j, carry):
        _half(2 * j, 0)
        _half(2 * j + 1, 1)
        return carry

    lax.fori_loop(0, (n + 2) // 2, pair, 0)
    npar2 = lax.rem(n, 2)
    for b in range(2):
        @pl.when((n >= 2) & (npar2 == b))
        def _df1(b=b):
            pltpu.make_async_copy(h_hbm.at[pl.ds(0, CH)], rows[b],
                                  ssem[b]).wait()

        @pl.when((n >= 1) & (npar2 == 1 - b))
        def _df2(b=b):
            pltpu.make_async_copy(h_hbm.at[pl.ds(0, CH)], rows[b],
                                  ssem[b]).wait()
    plsc.subcore_barrier()

    wds = [pltpu.async_copy(agg_sh.at[pl.ds(r0, 64)],
                            out_hbm.at[c, pl.ds(r0, 64)], zsem),
           pltpu.async_copy(agg_sh.at[pl.ds(r0 + 64, 8)],
                            out_hbm.at[c, pl.ds(r0 + 64, 8)], zsem)]
    for d in wds:
        d.wait()


def _prep_body(feat_ref, dsp_ref, ddp_ref, h1s_ref, io_ref, ii_ref):
    dsrc = jnp.sum(dsp_ref[...], axis=0)
    ddst = jnp.sum(ddp_ref[...], axis=0)
    inv_out = lax.rsqrt(jnp.maximum(dsrc, 1.0))
    inv_in = lax.rsqrt(jnp.maximum(ddst, 1.0))
    io_ref[...] = inv_out[:, None]
    ii_ref[...] = inv_in[:, None]
    h1s_ref[...] = feat_ref[...] * inv_out[:, None]


_prep = pl.pallas_call(
    _prep_body,
    out_shape=(
        jax.ShapeDtypeStruct((NPAD, D), jnp.float32),
        jax.ShapeDtypeStruct((NPAD, 1), jnp.float32),
        jax.ShapeDtypeStruct((NPAD, 1), jnp.float32),
    ),
)


_RB = 1280


def _mm1_body(p_ref, ii_ref, io_ref, w_ref, b_ref, out_ref):
    agg = (p_ref[0] + p_ref[1]) * ii_ref[...]
    z = jnp.dot(agg, w_ref[...], preferred_element_type=jnp.float32)
    z = z + b_ref[...]
    h = jnp.where(z > 0, z, 0.01 * z)
    out_ref[...] = h * io_ref[...]


_mm1 = pl.pallas_call(
    _mm1_body,
    grid=(NPAD // _RB,),
    in_specs=[
        pl.BlockSpec((2, _RB, D), lambda i: (0, i, 0)),
        pl.BlockSpec((_RB, 1), lambda i: (i, 0)),
        pl.BlockSpec((_RB, 1), lambda i: (i, 0)),
        pl.BlockSpec((D, D), lambda i: (0, 0)),
        pl.BlockSpec((1, D), lambda i: (0, 0)),
    ],
    out_specs=pl.BlockSpec((_RB, D), lambda i: (i, 0)),
    out_shape=jax.ShapeDtypeStruct((NPAD, D), jnp.float32),
)


def _fin_body(p_ref, ii_ref, w2_ref, b2_ref, wl_ref, bl_ref, s_ref, out_ref):
    acc = p_ref[0, 0]
    for ci in range(2):
        for ri in range(REP):
            if (ci, ri) != (0, 0):
                acc = acc + p_ref[ci, ri]
    agg = acc * ii_ref[...]
    z = jnp.dot(agg, w2_ref[...], preferred_element_type=jnp.float32)
    z = z + b2_ref[...]
    emb = jnp.where(z > 0, z, 0.01 * z)
    pooled = jnp.sum(emb, axis=0, keepdims=True) * s_ref[...]
    out_ref[...] = (
        jnp.dot(pooled, wl_ref[...], preferred_element_type=jnp.float32)
        + bl_ref[...])


_fin = pl.pallas_call(
    _fin_body,
    out_shape=jax.ShapeDtypeStruct((1, D), jnp.float32),
)


def kernel(feat, edge_index, order, W1, b1, W2, b2, Wl, bl):
    src = edge_index[0]
    dst = edge_index[1]
    featp = jnp.pad(feat, ((0, NPAD - N), (0, 0)))

    pk = (src << 14) | dst
    dsp, ddp, csrc, cdst, cnts = _stage_a(src, dst)
    h1s, inv_out, inv_in = _prep(featp, dsp, ddp)
    p1 = _scat1(h1s, pk)
    h2s = _mm1(p1, inv_in, inv_out, W1, b1.reshape(1, D))
    p2 = _scat2(h2s, csrc.reshape(-1), cdst.reshape(-1), cnts)
    scale = jnp.ones((1, D), jnp.float32) / (
        jnp.asarray(order, jnp.float32) + 1.0)
    p2r = p2.reshape(2, REP, A2BLK, D)[:, :, :P2]
    out = _fin(p2r, inv_in[:P2], W2, b2.reshape(1, D), Wl,
               bl.reshape(1, D), scale)
    return out.reshape(D)
